# Initial kernel scaffold; baseline (speedup 1.0000x reference)
import jax
import jax.numpy as jnp
from jax.experimental import pallas as pl


def kernel(x, edge_index, W1, b1, W2, b2):
    raise NotImplementedError("write your pallas kernel here")


# speedup vs baseline: 4.0647x; 4.0647x over previous
import functools

import jax
import jax.numpy as jnp
from jax import lax
from jax.experimental import pallas as pl
from jax.experimental.pallas import tpu as pltpu
from jax.experimental.pallas import tpu_sc as plsc

NC = 2
NS = 16
NW = NC * NS
L = 16
K = 128


def _sc_degrees(srcp, dstp, npa):
    ep = srcp.shape[0]
    ept = ep // NW
    nb = ept // K
    stripe = npa // NS
    mesh = plsc.VectorSubcoreMesh(core_axis_name="c", subcore_axis_name="s")

    def body(srcp_ref, dstp_ref, out_ref, deg_o, deg_i, sbuf, dbuf, ones, zv):
        c = lax.axis_index("c")
        s = lax.axis_index("s")
        zero16 = jnp.zeros((L,), jnp.float32)
        one16 = jnp.ones((L,), jnp.float32)

        def z_body(i, carry):
            zv[pl.ds(i * L, L)] = zero16
            return carry
        lax.fori_loop(0, stripe // L, z_body, 0)
        for j in range(K // L):
            ones[pl.ds(j * L, L)] = one16
        pltpu.sync_copy(zv, deg_o.at[pl.ds(s * stripe, stripe)])
        pltpu.sync_copy(zv, deg_i.at[pl.ds(s * stripe, stripe)])
        plsc.subcore_barrier()

        base = (c * NS + s) * ept

        def e_body(i, carry):
            off = base + i * K
            pltpu.sync_copy(srcp_ref.at[pl.ds(off, K)], sbuf)
            pltpu.sync_copy(dstp_ref.at[pl.ds(off, K)], dbuf)
            pltpu.sync_copy(ones, deg_o.at[sbuf], add=True)
            pltpu.sync_copy(ones, deg_i.at[dbuf], add=True)
            return carry
        lax.fori_loop(0, nb, e_body, 0)
        plsc.subcore_barrier()

        pltpu.sync_copy(deg_o.at[pl.ds(s * stripe, stripe)],
                        out_ref.at[2 * c, pl.ds(s * stripe, stripe)])
        pltpu.sync_copy(deg_i.at[pl.ds(s * stripe, stripe)],
                        out_ref.at[2 * c + 1, pl.ds(s * stripe, stripe)])

    k = pl.kernel(
        body,
        out_type=jax.ShapeDtypeStruct((4, npa), jnp.float32),
        mesh=mesh,
        scratch_types=[
            pltpu.VMEM_SHARED((npa,), jnp.float32),
            pltpu.VMEM_SHARED((npa,), jnp.float32),
            pltpu.VMEM((K,), jnp.int32),
            pltpu.VMEM((K,), jnp.int32),
            pltpu.VMEM((K,), jnp.float32),
            pltpu.VMEM((npa // NS,), jnp.float32),
        ],
    )
    return k(srcp, dstp)


def _sc_aggregate(xs, srcp, dstp):
    npa, d = xs.shape
    ep = srcp.shape[0]
    ept = ep // NW
    nb = ept // K
    stripe = npa // NS
    mesh = plsc.VectorSubcoreMesh(core_axis_name="c", subcore_axis_name="s")

    def body(xs_ref, srcp_ref, dstp_ref, out_ref, agg, sbuf, dbuf, rows, sem):
        c = lax.axis_index("c")
        s = lax.axis_index("s")
        zero16 = jnp.zeros((L,), jnp.float32)

        def z_body(i, carry):
            for j in range(d // L):
                rows[i, pl.ds(j * L, L)] = zero16
            return carry
        lax.fori_loop(0, K, z_body, 0)
        for t in range(stripe // K):
            pltpu.sync_copy(rows, agg.at[pl.ds(s * stripe + t * K, K)])
        plsc.subcore_barrier()

        base = (c * NS + s) * ept

        def e_body(i, carry):
            off = base + i * K
            pltpu.sync_copy(srcp_ref.at[pl.ds(off, K)], sbuf)
            pltpu.sync_copy(dstp_ref.at[pl.ds(off, K)], dbuf)
            pltpu.async_copy(xs_ref.at[sbuf], rows, sem).wait()
            pltpu.sync_copy(rows, agg.at[dbuf], add=True)
            return carry
        lax.fori_loop(0, nb, e_body, 0)
        plsc.subcore_barrier()

        pltpu.sync_copy(agg.at[pl.ds(s * stripe, stripe)],
                        out_ref.at[c, pl.ds(s * stripe, stripe)])

    k = pl.kernel(
        body,
        out_type=jax.ShapeDtypeStruct((NC, npa, d), jnp.float32),
        mesh=mesh,
        scratch_types=[
            pltpu.VMEM_SHARED((npa, d), jnp.float32),
            pltpu.VMEM((K,), jnp.int32),
            pltpu.VMEM((K,), jnp.int32),
            pltpu.VMEM((K, d), jnp.float32),
            pltpu.SemaphoreType.DMA,
        ],
    )
    return k(xs, srcp, dstp)


def _tc_prescale(degs_t, x_pad, block):
    npa, d = x_pad.shape

    def body(dg_ref, x_ref, xs_ref, nm_ref):
        dg = dg_ref[...]
        do = dg[:, 0:1] + dg[:, 2:3]
        di = dg[:, 1:2] + dg[:, 3:4]
        ns = lax.rsqrt(jnp.where(do > 0, do, 1.0))
        nd = lax.rsqrt(jnp.where(di > 0, di, 1.0))
        xs_ref[...] = x_ref[...] * ns
        nm_ref[...] = jnp.concatenate([ns, nd], axis=1)

    return pl.pallas_call(
        body,
        grid=(npa // block,),
        in_specs=[
            pl.BlockSpec((block, 4), lambda i: (i, 0)),
            pl.BlockSpec((block, d), lambda i: (i, 0)),
        ],
        out_specs=[
            pl.BlockSpec((block, d), lambda i: (i, 0)),
            pl.BlockSpec((block, 2), lambda i: (i, 0)),
        ],
        out_shape=[
            jax.ShapeDtypeStruct((npa, d), jnp.float32),
            jax.ShapeDtypeStruct((npa, 2), jnp.float32),
        ],
    )(degs_t, x_pad)


def _tc_dense(aggp, norms, w, b, relu, scale_out, block):
    _, npa, d = aggp.shape

    def body(agg_ref, nm_ref, w_ref, b_ref, out_ref):
        agg = agg_ref[0] + agg_ref[1]
        a = agg * nm_ref[:, 1:2]
        h = jnp.dot(a, w_ref[...], preferred_element_type=jnp.float32,
                    precision=lax.Precision.HIGHEST) + b_ref[...]
        if relu:
            h = jnp.maximum(h, 0.0)
        if scale_out:
            h = h * nm_ref[:, 0:1]
        out_ref[...] = h

    return pl.pallas_call(
        body,
        grid=(npa // block,),
        in_specs=[
            pl.BlockSpec((2, block, d), lambda i: (0, i, 0)),
            pl.BlockSpec((block, 2), lambda i: (i, 0)),
            pl.BlockSpec((d, d), lambda i: (0, 0)),
            pl.BlockSpec((1, d), lambda i: (0, 0)),
        ],
        out_specs=pl.BlockSpec((block, d), lambda i: (i, 0)),
        out_shape=jax.ShapeDtypeStruct((npa, d), jnp.float32),
    )(aggp, norms, w, b)


def kernel(x, edge_index, W1, b1, W2, b2):
    n, d = x.shape
    e = edge_index.shape[1]
    npa = 10240
    assert n < npa and d % L == 0 and npa % (NS * K) == 0

    src = edge_index[0].astype(jnp.int32)
    dst = edge_index[1].astype(jnp.int32)
    ep = -(-e // (NW * K)) * (NW * K)
    pad = ep - e
    fill = jnp.full((pad,), n, jnp.int32)
    srcp = jnp.concatenate([src, fill])
    dstp = jnp.concatenate([dst, fill])
    x_pad = jnp.pad(x, ((0, npa - n), (0, 0)))

    degs = _sc_degrees(srcp, dstp, npa)
    degs_t = degs.T
    block = 512
    xs, norms = _tc_prescale(degs_t, x_pad, block)
    aggp1 = _sc_aggregate(xs, srcp, dstp)
    h1s = _tc_dense(aggp1, norms, W1, b1.reshape(1, d), True, True, block)
    aggp2 = _sc_aggregate(h1s, srcp, dstp)
    out = _tc_dense(aggp2, norms, W2, b2.reshape(1, d), False, False, block)
    return out[:n]


# speedup vs baseline: 4.3670x; 1.0744x over previous
import functools

import jax
import jax.numpy as jnp
from jax import lax
from jax.experimental import pallas as pl
from jax.experimental.pallas import tpu as pltpu
from jax.experimental.pallas import tpu_sc as plsc

NC = 2
NS = 16
NW = NC * NS
L = 16
K = 128
RING = 2


def _sc_degrees(src3, dst3, npa):
    _, nb, _ = src3.shape
    stripe = npa // NS
    mesh = plsc.VectorSubcoreMesh(core_axis_name="c", subcore_axis_name="s")

    def body(src3_ref, dst3_ref, out_ref, deg_o, deg_i, sbufs, dbufs, ones, zv):
        c = lax.axis_index("c")
        s = lax.axis_index("s")
        zero16 = jnp.zeros((L,), jnp.float32)
        one16 = jnp.ones((L,), jnp.float32)

        def z_body(i, carry):
            zv[pl.ds(i * L, L)] = zero16
            return carry
        lax.fori_loop(0, stripe // L, z_body, 0)
        for j in range(K // L):
            ones[pl.ds(j * L, L)] = one16
        pltpu.sync_copy(zv, deg_o.at[pl.ds(s * stripe, stripe)])
        pltpu.sync_copy(zv, deg_i.at[pl.ds(s * stripe, stripe)])

        wid = c * NS + s
        pltpu.sync_copy(src3_ref.at[wid], sbufs)
        pltpu.sync_copy(dst3_ref.at[wid], dbufs)
        plsc.subcore_barrier()

        def e_body(i, carry):
            pltpu.sync_copy(ones, deg_o.at[sbufs.at[i]], add=True)
            pltpu.sync_copy(ones, deg_i.at[dbufs.at[i]], add=True)
            return carry
        lax.fori_loop(0, nb, e_body, 0)
        plsc.subcore_barrier()

        pltpu.sync_copy(deg_o.at[pl.ds(s * stripe, stripe)],
                        out_ref.at[2 * c, pl.ds(s * stripe, stripe)])
        pltpu.sync_copy(deg_i.at[pl.ds(s * stripe, stripe)],
                        out_ref.at[2 * c + 1, pl.ds(s * stripe, stripe)])

    k = pl.kernel(
        body,
        out_type=jax.ShapeDtypeStruct((4, npa), jnp.float32),
        mesh=mesh,
        scratch_types=[
            pltpu.VMEM_SHARED((npa,), jnp.float32),
            pltpu.VMEM_SHARED((npa,), jnp.float32),
            pltpu.VMEM((nb, K), jnp.int32),
            pltpu.VMEM((nb, K), jnp.int32),
            pltpu.VMEM((K,), jnp.float32),
            pltpu.VMEM((npa // NS,), jnp.float32),
        ],
    )
    return k(src3, dst3)


def _sc_aggregate(xs, src3, dst3):
    npa, d = xs.shape
    _, nb, _ = src3.shape
    stripe = npa // NS
    mesh = plsc.VectorSubcoreMesh(core_axis_name="c", subcore_axis_name="s")

    def body(xs_ref, src3_ref, dst3_ref, out_ref, agg, sbufs, dbufr,
             rows0, rows1, sem0, sem1):
        c = lax.axis_index("c")
        s = lax.axis_index("s")
        zero16 = jnp.zeros((L,), jnp.float32)
        rows = (rows0, rows1)
        sems = (sem0, sem1)

        def z_body(i, carry):
            for j in range(d // L):
                rows0[i, pl.ds(j * L, L)] = zero16
            return carry
        lax.fori_loop(0, K, z_body, 0)
        for t in range(stripe // K):
            pltpu.sync_copy(rows0, agg.at[pl.ds(s * stripe + t * K, K)])

        wid = c * NS + s
        pltpu.sync_copy(src3_ref.at[wid], sbufs)
        plsc.subcore_barrier()

        for b in range(RING):
            pltpu.async_copy(xs_ref.at[sbufs.at[b]], rows[b], sems[b])
            pltpu.async_copy(dst3_ref.at[wid].at[b], dbufr.at[b], sems[b])

        def e_body(g, carry):
            for b in range(RING):
                i = g * RING + b
                pltpu.make_async_copy(xs_ref.at[sbufs.at[i]], rows[b],
                                      sems[b]).wait()
                pltpu.make_async_copy(dst3_ref.at[wid].at[i], dbufr.at[b],
                                      sems[b]).wait()
                pltpu.sync_copy(rows[b], agg.at[dbufr.at[b]], add=True)
                pltpu.async_copy(xs_ref.at[sbufs.at[i + RING]], rows[b],
                                 sems[b])
                pltpu.async_copy(dst3_ref.at[wid].at[i + RING], dbufr.at[b],
                                 sems[b])
            return carry
        lax.fori_loop(0, nb // RING - 1, e_body, 0)

        for b in range(RING):
            i = nb - RING + b
            pltpu.make_async_copy(xs_ref.at[sbufs.at[i]], rows[b],
                                  sems[b]).wait()
            pltpu.make_async_copy(dst3_ref.at[wid].at[i], dbufr.at[b],
                                  sems[b]).wait()
            pltpu.sync_copy(rows[b], agg.at[dbufr.at[b]], add=True)
        plsc.subcore_barrier()

        pltpu.sync_copy(agg.at[pl.ds(s * stripe, stripe)],
                        out_ref.at[c, pl.ds(s * stripe, stripe)])

    k = pl.kernel(
        body,
        out_type=jax.ShapeDtypeStruct((NC, npa, d), jnp.float32),
        mesh=mesh,
        scratch_types=[
            pltpu.VMEM_SHARED((npa, d), jnp.float32),
            pltpu.VMEM((nb, K), jnp.int32),
            pltpu.VMEM((RING, K), jnp.int32),
            pltpu.VMEM((K, d), jnp.float32),
            pltpu.VMEM((K, d), jnp.float32),
            pltpu.SemaphoreType.DMA,
            pltpu.SemaphoreType.DMA,
        ],
    )
    return k(xs, src3, dst3)


def _tc_prescale(degs_t, x_pad, block):
    npa, d = x_pad.shape

    def body(dg_ref, x_ref, xs_ref, nm_ref):
        dg = dg_ref[...]
        do = dg[:, 0:1] + dg[:, 2:3]
        di = dg[:, 1:2] + dg[:, 3:4]
        ns = lax.rsqrt(jnp.where(do > 0, do, 1.0))
        nd = lax.rsqrt(jnp.where(di > 0, di, 1.0))
        xs_ref[...] = x_ref[...] * ns
        nm_ref[...] = jnp.concatenate([ns, nd], axis=1)

    return pl.pallas_call(
        body,
        grid=(npa // block,),
        in_specs=[
            pl.BlockSpec((block, 4), lambda i: (i, 0)),
            pl.BlockSpec((block, d), lambda i: (i, 0)),
        ],
        out_specs=[
            pl.BlockSpec((block, d), lambda i: (i, 0)),
            pl.BlockSpec((block, 2), lambda i: (i, 0)),
        ],
        out_shape=[
            jax.ShapeDtypeStruct((npa, d), jnp.float32),
            jax.ShapeDtypeStruct((npa, 2), jnp.float32),
        ],
    )(degs_t, x_pad)


def _tc_dense(aggp, norms, w, b, relu, scale_out, block):
    _, npa, d = aggp.shape

    def body(agg_ref, nm_ref, w_ref, b_ref, out_ref):
        agg = agg_ref[0] + agg_ref[1]
        a = agg * nm_ref[:, 1:2]
        h = jnp.dot(a, w_ref[...], preferred_element_type=jnp.float32,
                    precision=lax.Precision.HIGHEST) + b_ref[...]
        if relu:
            h = jnp.maximum(h, 0.0)
        if scale_out:
            h = h * nm_ref[:, 0:1]
        out_ref[...] = h

    return pl.pallas_call(
        body,
        grid=(npa // block,),
        in_specs=[
            pl.BlockSpec((2, block, d), lambda i: (0, i, 0)),
            pl.BlockSpec((block, 2), lambda i: (i, 0)),
            pl.BlockSpec((d, d), lambda i: (0, 0)),
            pl.BlockSpec((1, d), lambda i: (0, 0)),
        ],
        out_specs=pl.BlockSpec((block, d), lambda i: (i, 0)),
        out_shape=jax.ShapeDtypeStruct((npa, d), jnp.float32),
    )(aggp, norms, w, b)


def kernel(x, edge_index, W1, b1, W2, b2):
    n, d = x.shape
    e = edge_index.shape[1]
    npa = 10240
    assert n < npa and d % L == 0 and npa % (NS * K) == 0

    src = edge_index[0].astype(jnp.int32)
    dst = edge_index[1].astype(jnp.int32)
    chunk = NW * K * RING
    ep = -(-e // chunk) * chunk
    pad = ep - e
    fill = jnp.full((pad,), n, jnp.int32)
    nb = ep // (NW * K)
    src3 = jnp.concatenate([src, fill]).reshape(NW, nb, K)
    dst3 = jnp.concatenate([dst, fill]).reshape(NW, nb, K)
    x_pad = jnp.pad(x, ((0, npa - n), (0, 0)))

    degs = _sc_degrees(src3, dst3, npa)
    degs_t = degs.T
    block = 512
    xs, norms = _tc_prescale(degs_t, x_pad, block)
    aggp1 = _sc_aggregate(xs, src3, dst3)
    h1s = _tc_dense(aggp1, norms, W1, b1.reshape(1, d), True, True, block)
    aggp2 = _sc_aggregate(h1s, src3, dst3)
    out = _tc_dense(aggp2, norms, W2, b2.reshape(1, d), False, False, block)
    return out[:n]


# speedup vs baseline: 12.2610x; 2.8077x over previous
import functools

import jax
import jax.numpy as jnp
from jax import lax
from jax.experimental import pallas as pl
from jax.experimental.pallas import tpu as pltpu
from jax.experimental.pallas import tpu_sc as plsc

NC = 2
NS = 16
NW = NC * NS
L = 16
K = 128
RING = 2


def _sc_degrees(src3, dst3, npa):
    _, nb, _ = src3.shape
    stripe = npa // NS
    mesh = plsc.VectorSubcoreMesh(core_axis_name="c", subcore_axis_name="s")

    def body(src3_ref, dst3_ref, out_ref, deg_o, deg_i, sbufs, dbufs, ones, zv):
        c = lax.axis_index("c")
        s = lax.axis_index("s")
        zero16 = jnp.zeros((L,), jnp.float32)
        one16 = jnp.ones((L,), jnp.float32)

        def z_body(i, carry):
            zv[pl.ds(i * L, L)] = zero16
            return carry
        lax.fori_loop(0, stripe // L, z_body, 0)
        for j in range(K // L):
            ones[pl.ds(j * L, L)] = one16
        pltpu.sync_copy(zv, deg_o.at[pl.ds(s * stripe, stripe)])
        pltpu.sync_copy(zv, deg_i.at[pl.ds(s * stripe, stripe)])

        wid = c * NS + s
        pltpu.sync_copy(src3_ref.at[wid], sbufs)
        pltpu.sync_copy(dst3_ref.at[wid], dbufs)
        plsc.subcore_barrier()

        def e_body(i, carry):
            pltpu.sync_copy(ones, deg_o.at[sbufs.at[i]], add=True)
            pltpu.sync_copy(ones, deg_i.at[dbufs.at[i]], add=True)
            return carry
        lax.fori_loop(0, nb, e_body, 0)
        plsc.subcore_barrier()

        pltpu.sync_copy(deg_o.at[pl.ds(s * stripe, stripe)],
                        out_ref.at[2 * c, pl.ds(s * stripe, stripe)])
        pltpu.sync_copy(deg_i.at[pl.ds(s * stripe, stripe)],
                        out_ref.at[2 * c + 1, pl.ds(s * stripe, stripe)])

    k = pl.kernel(
        body,
        out_type=jax.ShapeDtypeStruct((4, npa), jnp.float32),
        mesh=mesh,
        scratch_types=[
            pltpu.VMEM_SHARED((npa,), jnp.float32),
            pltpu.VMEM_SHARED((npa,), jnp.float32),
            pltpu.VMEM((nb, K), jnp.int32),
            pltpu.VMEM((nb, K), jnp.int32),
            pltpu.VMEM((K,), jnp.float32),
            pltpu.VMEM((npa // NS,), jnp.float32),
        ],
    )
    return k(src3, dst3)


def _sc_aggregate(xs, src3, dst3):
    npa, d = xs.shape
    _, nb, _ = src3.shape
    stripe = npa // NS
    mesh = plsc.VectorSubcoreMesh(core_axis_name="c", subcore_axis_name="s")

    def body(xs_ref, src3_ref, dst3_ref, out_ref, agg, sbufs, dbufr,
             rows0, rows1, sem0, sem1):
        c = lax.axis_index("c")
        s = lax.axis_index("s")
        zero16 = jnp.zeros((L,), jnp.float32)
        rows = (rows0, rows1)
        sems = (sem0, sem1)

        def z_body(i, carry):
            for j in range(d // L):
                rows0[i, pl.ds(j * L, L)] = zero16
            return carry
        lax.fori_loop(0, K, z_body, 0)
        for t in range(stripe // K):
            pltpu.sync_copy(rows0, agg.at[pl.ds(s * stripe + t * K, K)])

        wid = c * NS + s
        pltpu.sync_copy(src3_ref.at[wid], sbufs)
        plsc.subcore_barrier()

        for b in range(RING):
            pltpu.async_copy(xs_ref.at[sbufs.at[b]], rows[b], sems[b])
            pltpu.async_copy(dst3_ref.at[wid].at[b], dbufr.at[b], sems[b])

        def e_body(g, carry):
            for b in range(RING):
                i = g * RING + b
                pltpu.make_async_copy(xs_ref.at[sbufs.at[i]], rows[b],
                                      sems[b]).wait()
                pltpu.make_async_copy(dst3_ref.at[wid].at[i], dbufr.at[b],
                                      sems[b]).wait()
                pltpu.sync_copy(rows[b], agg.at[dbufr.at[b]], add=True)
                pltpu.async_copy(xs_ref.at[sbufs.at[i + RING]], rows[b],
                                 sems[b])
                pltpu.async_copy(dst3_ref.at[wid].at[i + RING], dbufr.at[b],
                                 sems[b])
            return carry
        lax.fori_loop(0, nb // RING - 1, e_body, 0)

        for b in range(RING):
            i = nb - RING + b
            pltpu.make_async_copy(xs_ref.at[sbufs.at[i]], rows[b],
                                  sems[b]).wait()
            pltpu.make_async_copy(dst3_ref.at[wid].at[i], dbufr.at[b],
                                  sems[b]).wait()
            pltpu.sync_copy(rows[b], agg.at[dbufr.at[b]], add=True)
        plsc.subcore_barrier()

        pltpu.sync_copy(agg.at[pl.ds(s * stripe, stripe)],
                        out_ref.at[c, pl.ds(s * stripe, stripe)])

    k = pl.kernel(
        body,
        out_type=jax.ShapeDtypeStruct((NC, npa, d), jnp.float32),
        mesh=mesh,
        scratch_types=[
            pltpu.VMEM_SHARED((npa, d), jnp.float32),
            pltpu.VMEM((nb, K), jnp.int32),
            pltpu.VMEM((RING, K), jnp.int32),
            pltpu.VMEM((K, d), jnp.float32),
            pltpu.VMEM((K, d), jnp.float32),
            pltpu.SemaphoreType.DMA,
            pltpu.SemaphoreType.DMA,
        ],
    )
    return k(xs, src3, dst3)


def _tc_prescale(degs_t, x_pad, block):
    npa, d = x_pad.shape

    def body(dg_ref, x_ref, xs_ref, nm_ref):
        dg = dg_ref[...]
        do = dg[:, 0:1] + dg[:, 2:3]
        di = dg[:, 1:2] + dg[:, 3:4]
        ns = lax.rsqrt(jnp.where(do > 0, do, 1.0))
        nd = lax.rsqrt(jnp.where(di > 0, di, 1.0))
        xs_ref[...] = x_ref[...] * ns
        nm_ref[...] = jnp.concatenate([ns, nd], axis=1)

    return pl.pallas_call(
        body,
        grid=(npa // block,),
        in_specs=[
            pl.BlockSpec((block, 4), lambda i: (i, 0)),
            pl.BlockSpec((block, d), lambda i: (i, 0)),
        ],
        out_specs=[
            pl.BlockSpec((block, d), lambda i: (i, 0)),
            pl.BlockSpec((block, 2), lambda i: (i, 0)),
        ],
        out_shape=[
            jax.ShapeDtypeStruct((npa, d), jnp.float32),
            jax.ShapeDtypeStruct((npa, 2), jnp.float32),
        ],
    )(degs_t, x_pad)


def _tc_dense(aggp, norms, w, b, relu, scale_out, block):
    _, npa, d = aggp.shape

    def body(agg_ref, nm_ref, w_ref, b_ref, out_ref):
        agg = agg_ref[0] + agg_ref[1]
        a = agg * nm_ref[:, 1:2]
        h = jnp.dot(a, w_ref[...], preferred_element_type=jnp.float32,
                    precision=lax.Precision.HIGHEST) + b_ref[...]
        if relu:
            h = jnp.maximum(h, 0.0)
        if scale_out:
            h = h * nm_ref[:, 0:1]
        out_ref[...] = h

    return pl.pallas_call(
        body,
        grid=(npa // block,),
        in_specs=[
            pl.BlockSpec((2, block, d), lambda i: (0, i, 0)),
            pl.BlockSpec((block, 2), lambda i: (i, 0)),
            pl.BlockSpec((d, d), lambda i: (0, 0)),
            pl.BlockSpec((1, d), lambda i: (0, 0)),
        ],
        out_specs=pl.BlockSpec((block, d), lambda i: (i, 0)),
        out_shape=jax.ShapeDtypeStruct((npa, d), jnp.float32),
    )(aggp, norms, w, b)


def kernel(x, edge_index, W1, b1, W2, b2):
    n, d = x.shape
    e = edge_index.shape[1]
    npa = 10240
    assert n < npa and d % L == 0 and npa % (NS * K) == 0

    src = edge_index[0].astype(jnp.int32)
    dst = edge_index[1].astype(jnp.int32)
    chunk = NW * K * RING
    ep = -(-e // chunk) * chunk
    pad = ep - e
    fill = n + jnp.arange(pad, dtype=jnp.int32) % (npa - n)
    nb = ep // (NW * K)
    src3 = jnp.concatenate([src, fill]).reshape(NW, nb, K)
    dst3 = jnp.concatenate([dst, fill]).reshape(NW, nb, K)
    x_pad = jnp.pad(x, ((0, npa - n), (0, 0)))

    degs = _sc_degrees(src3, dst3, npa)
    degs_t = degs.T
    block = 512
    xs, norms = _tc_prescale(degs_t, x_pad, block)
    aggp1 = _sc_aggregate(xs, src3, dst3)
    h1s = _tc_dense(aggp1, norms, W1, b1.reshape(1, d), True, True, block)
    aggp2 = _sc_aggregate(h1s, src3, dst3)
    out = _tc_dense(aggp2, norms, W2, b2.reshape(1, d), False, False, block)
    return out[:n]
